# baseline (speedup 1.0000x reference)
import functools

import jax
import jax.numpy as jnp
from jax import lax
from jax.experimental import pallas as pl
from jax.experimental.pallas import tpu as pltpu
from jax.experimental.pallas import tpu_sc as plsc

_N_TABLES = 26
_VOCAB = 100000
_DIM = 64
_BATCH = 4096
_MH = 3

_NC, _NS, _L = 2, 16, 16
_NW = _NC * _NS
_CB = 32
_NQ = _BATCH // _CB
_QW = _NQ // _NW
_GR = _CB * _MH
_NB = 2
_OD = _N_TABLES * _DIM


def _sc_embedding_bag(g_index, g_base, tables2):
    mesh = plsc.VectorSubcoreMesh(
        core_axis_name="c", subcore_axis_name="s",
        num_cores=_NC, num_subcores=_NS,
    )

    @functools.partial(
        pl.kernel,
        out_type=jax.ShapeDtypeStruct((_BATCH, _OD), jnp.float32),
        mesh=mesh,
        compiler_params=pltpu.CompilerParams(use_tc_tiling_on_sc=True),
        scratch_types=[
            pltpu.VMEM((2, _N_TABLES, _GR), jnp.int32),
            pltpu.VMEM((2, _N_TABLES, _CB * _L), jnp.int32),
            pltpu.VMEM((_NB, _GR, 128), jnp.float32),
            pltpu.VMEM((_CB, _OD), jnp.float32),
            pltpu.SemaphoreType.DMA((_NB,)),
            pltpu.SemaphoreType.DMA((2,)),
            pltpu.SemaphoreType.DMA((2,)),
        ],
    )
    def k(idx_hbm, base_hbm, tbl_hbm, out_hbm,
          idx_v, base_v, rows_v, out_v, sem, isem, bsem):
        wid = lax.axis_index("s") * _NC + lax.axis_index("c")

        def idx_fetch(qi, slot):
            q = wid * _QW + qi
            pltpu.async_copy(idx_hbm.at[q], idx_v.at[slot], isem.at[slot])
            pltpu.async_copy(base_hbm.at[q], base_v.at[slot], bsem.at[slot])

        def idx_wait(slot):
            pltpu.make_async_copy(
                idx_hbm.at[0], idx_v.at[slot], isem.at[slot]
            ).wait()
            pltpu.make_async_copy(
                base_hbm.at[0], base_v.at[slot], bsem.at[slot]
            ).wait()

        def gather(slot, t, buf):
            pltpu.async_copy(
                tbl_hbm.at[idx_v.at[slot, t]], rows_v.at[buf], sem.at[buf]
            )

        def drain(buf):
            pltpu.make_async_copy(
                tbl_hbm.at[idx_v.at[0, 0]], rows_v.at[buf], sem.at[buf]
            ).wait()

        def pool(slot, t, buf):
            def bag(b, c2):
                r = b * _MH
                pv = base_v[slot, t, pl.ds(b * _L, _L)]
                b0, b1, b2 = pv[0], pv[1], pv[2]
                for kk in range(_DIM // _L):
                    o = kk * _L
                    out_v[b, pl.ds(t * _DIM + o, _L)] = (
                        rows_v[buf, r, pl.ds(b0 + o, _L)]
                        + rows_v[buf, r + 1, pl.ds(b1 + o, _L)]
                        + rows_v[buf, r + 2, pl.ds(b2 + o, _L)]
                    )
                return c2

            lax.fori_loop(0, _CB, bag, 0)

        idx_fetch(0, 0)
        for qi in range(_QW):
            slot = qi % 2
            idx_wait(slot)
            if qi + 1 < _QW:
                idx_fetch(qi + 1, (qi + 1) % 2)
            gather(slot, 0, 0)

            def table(t, c1):
                buf = t & 1
                gather(slot, jnp.minimum(t + 1, _N_TABLES - 1), 1 - buf)
                drain(buf)
                pool(slot, t, buf)
                return c1

            lax.fori_loop(0, _N_TABLES, table, 0)
            drain(0)
            pltpu.sync_copy(
                out_v, out_hbm.at[pl.ds((wid * _QW + qi) * _CB, _CB)]
            )

    return k(g_index, g_base, tables2)


def kernel(index, offset, tables):
    del offset
    rfull = index + (jnp.arange(_N_TABLES, dtype=jnp.int32) * _VOCAB)[:, None]
    j = (rfull >> 1).reshape(_N_TABLES, _NQ, _GR).transpose(1, 0, 2)
    p64 = ((rfull & 1) << 6).reshape(_N_TABLES, _NQ, _CB, _MH)
    p64 = jnp.pad(p64, ((0, 0), (0, 0), (0, 0), (0, _L - _MH)))
    p64 = p64.transpose(1, 0, 2, 3).reshape(_NQ, _N_TABLES, _CB * _L)
    tables2 = tables.reshape(_N_TABLES * _VOCAB * _DIM // 128, 128)
    out = _sc_embedding_bag(j, p64, tables2)
    return out.reshape(_BATCH, _N_TABLES, _DIM)


# speedup vs baseline: 1.0118x; 1.0118x over previous
import functools

import jax
import jax.numpy as jnp
from jax import lax
from jax.experimental import pallas as pl
from jax.experimental.pallas import tpu as pltpu
from jax.experimental.pallas import tpu_sc as plsc

_N_TABLES = 26
_VOCAB = 100000
_DIM = 64
_BATCH = 4096
_MH = 3

_NC, _NS, _L = 2, 16, 16
_NW = _NC * _NS
_CB = 32
_NQ = _BATCH // _CB
_QW = _NQ // _NW
_GR = _CB * _MH
_NB = 2


def _sc_embedding_bag(g_index, tables_flat):
    mesh = plsc.VectorSubcoreMesh(
        core_axis_name="c", subcore_axis_name="s",
        num_cores=_NC, num_subcores=_NS,
    )

    @functools.partial(
        pl.kernel,
        out_type=jax.ShapeDtypeStruct((_BATCH, _N_TABLES, _DIM), jnp.float32),
        mesh=mesh,
        compiler_params=pltpu.CompilerParams(use_tc_tiling_on_sc=False),
        scratch_types=[
            pltpu.VMEM((_N_TABLES, _GR), jnp.int32),
            pltpu.VMEM((_NB, _GR, _DIM), jnp.float32),
            pltpu.VMEM((_CB, _N_TABLES, _DIM), jnp.float32),
            pltpu.SemaphoreType.DMA((_NB,)),
        ],
    )
    def k(idx_hbm, tbl_hbm, out_hbm, idx_v, rows_v, out_v, sem):
        wid = lax.axis_index("s") * _NC + lax.axis_index("c")

        def gather(t, buf):
            pltpu.async_copy(
                tbl_hbm.at[idx_v.at[t]], rows_v.at[buf], sem.at[buf]
            )

        def drain(buf):
            pltpu.make_async_copy(
                tbl_hbm.at[idx_v.at[0]], rows_v.at[buf], sem.at[buf]
            ).wait()

        def pool(t, buf):
            @plsc.parallel_loop(0, _CB)
            def bag(b):
                r = b * _MH
                for kk in range(_DIM // _L):
                    sl = pl.ds(kk * _L, _L)
                    out_v[b, t, sl] = (
                        rows_v[buf, r, sl]
                        + rows_v[buf, r + 1, sl]
                        + rows_v[buf, r + 2, sl]
                    )

        def chunk(qi, carry):
            pltpu.sync_copy(idx_hbm.at[wid * _QW + qi], idx_v)
            for t in range(_NB - 1):
                gather(t, t)
            for t in range(_N_TABLES):
                buf = t % _NB
                drain(buf)
                pool(t, buf)
                if t + (_NB - 1) < _N_TABLES:
                    gather(t + (_NB - 1), (t + (_NB - 1)) % _NB)
            pltpu.sync_copy(
                out_v, out_hbm.at[pl.ds((wid * _QW + qi) * _CB, _CB)]
            )
            return carry

        lax.fori_loop(0, _QW, chunk, 0)

    return k(g_index, tables_flat)


def kernel(index, offset, tables):
    del offset
    g_index = index + (jnp.arange(_N_TABLES, dtype=jnp.int32) * _VOCAB)[:, None]
    g_index = g_index.reshape(_N_TABLES, _NQ, _GR).transpose(1, 0, 2)
    tables_flat = tables.reshape(_N_TABLES * _VOCAB, _DIM)
    return _sc_embedding_bag(g_index, tables_flat)


# speedup vs baseline: 1.0546x; 1.0424x over previous
import functools

import jax
import jax.numpy as jnp
from jax import lax
from jax.experimental import pallas as pl
from jax.experimental.pallas import tpu as pltpu
from jax.experimental.pallas import tpu_sc as plsc

_N_TABLES = 26
_VOCAB = 100000
_DIM = 64
_BATCH = 4096
_MH = 3

_NC, _NS, _L = 2, 16, 16
_NW = _NC * _NS
_CB = 32
_NQ = _BATCH // _CB
_QW = _NQ // _NW
_GR = _CB * _MH
_NB = 4


def _sc_embedding_bag(g_index, tables_flat):
    mesh = plsc.VectorSubcoreMesh(
        core_axis_name="c", subcore_axis_name="s",
        num_cores=_NC, num_subcores=_NS,
    )

    @functools.partial(
        pl.kernel,
        out_type=jax.ShapeDtypeStruct((_BATCH, _N_TABLES, _DIM), jnp.float32),
        mesh=mesh,
        compiler_params=pltpu.CompilerParams(use_tc_tiling_on_sc=False),
        scratch_types=[
            pltpu.VMEM((_N_TABLES, _GR), jnp.int32),
            pltpu.VMEM((_NB, _GR, _DIM), jnp.float32),
            pltpu.VMEM((_CB, _N_TABLES, _DIM), jnp.float32),
            pltpu.SemaphoreType.DMA((_NB,)),
        ],
    )
    def k(idx_hbm, tbl_hbm, out_hbm, idx_v, rows_v, out_v, sem):
        wid = lax.axis_index("s") * _NC + lax.axis_index("c")

        def gather(t, buf):
            pltpu.async_copy(
                tbl_hbm.at[idx_v.at[t]], rows_v.at[buf], sem.at[buf]
            )

        def drain(buf):
            pltpu.make_async_copy(
                tbl_hbm.at[idx_v.at[0]], rows_v.at[buf], sem.at[buf]
            ).wait()

        def pool(t, buf):
            @plsc.parallel_loop(0, _CB)
            def bag(b):
                r = b * _MH
                for kk in range(_DIM // _L):
                    sl = pl.ds(kk * _L, _L)
                    out_v[b, t, sl] = (
                        rows_v[buf, r, sl]
                        + rows_v[buf, r + 1, sl]
                        + rows_v[buf, r + 2, sl]
                    )

        def chunk(qi, carry):
            pltpu.sync_copy(idx_hbm.at[wid * _QW + qi], idx_v)
            for t in range(_NB - 1):
                gather(t, t)
            for t in range(_N_TABLES):
                buf = t % _NB
                drain(buf)
                pool(t, buf)
                if t + (_NB - 1) < _N_TABLES:
                    gather(t + (_NB - 1), (t + (_NB - 1)) % _NB)
            pltpu.sync_copy(
                out_v, out_hbm.at[pl.ds((wid * _QW + qi) * _CB, _CB)]
            )
            return carry

        lax.fori_loop(0, _QW, chunk, 0)

    return k(g_index, tables_flat)


def kernel(index, offset, tables):
    del offset
    g_index = index + (jnp.arange(_N_TABLES, dtype=jnp.int32) * _VOCAB)[:, None]
    g_index = g_index.reshape(_N_TABLES, _NQ, _GR).transpose(1, 0, 2)
    tables_flat = tables.reshape(_N_TABLES * _VOCAB, _DIM)
    return _sc_embedding_bag(g_index, tables_flat)


# speedup vs baseline: 1.0617x; 1.0067x over previous
import functools

import jax
import jax.numpy as jnp
from jax import lax
from jax.experimental import pallas as pl
from jax.experimental.pallas import tpu as pltpu
from jax.experimental.pallas import tpu_sc as plsc

_N_TABLES = 26
_VOCAB = 100000
_DIM = 64
_BATCH = 4096
_MH = 3

_NC, _NS, _L = 2, 16, 16
_NW = _NC * _NS
_CB = 32
_NQ = _BATCH // _CB
_QW = _NQ // _NW
_GR = _CB * _MH
_NB = 6


def _sc_embedding_bag(g_index, tables_flat):
    mesh = plsc.VectorSubcoreMesh(
        core_axis_name="c", subcore_axis_name="s",
        num_cores=_NC, num_subcores=_NS,
    )

    @functools.partial(
        pl.kernel,
        out_type=jax.ShapeDtypeStruct((_BATCH, _N_TABLES, _DIM), jnp.float32),
        mesh=mesh,
        compiler_params=pltpu.CompilerParams(use_tc_tiling_on_sc=False),
        scratch_types=[
            pltpu.VMEM((_N_TABLES, _GR), jnp.int32),
            pltpu.VMEM((_NB, _GR, _DIM), jnp.float32),
            pltpu.VMEM((_CB, _N_TABLES, _DIM), jnp.float32),
            pltpu.SemaphoreType.DMA((_NB,)),
        ],
    )
    def k(idx_hbm, tbl_hbm, out_hbm, idx_v, rows_v, out_v, sem):
        wid = lax.axis_index("s") * _NC + lax.axis_index("c")

        def gather(t, buf):
            pltpu.async_copy(
                tbl_hbm.at[idx_v.at[t]], rows_v.at[buf], sem.at[buf]
            )

        def drain(buf):
            pltpu.make_async_copy(
                tbl_hbm.at[idx_v.at[0]], rows_v.at[buf], sem.at[buf]
            ).wait()

        def pool(t, buf):
            @plsc.parallel_loop(0, _CB)
            def bag(b):
                r = b * _MH
                for kk in range(_DIM // _L):
                    sl = pl.ds(kk * _L, _L)
                    out_v[b, t, sl] = (
                        rows_v[buf, r, sl]
                        + rows_v[buf, r + 1, sl]
                        + rows_v[buf, r + 2, sl]
                    )

        def chunk(qi, carry):
            pltpu.sync_copy(idx_hbm.at[wid * _QW + qi], idx_v)
            for t in range(_NB - 1):
                gather(t, t)
            for t in range(_N_TABLES):
                buf = t % _NB
                drain(buf)
                pool(t, buf)
                if t + (_NB - 1) < _N_TABLES:
                    gather(t + (_NB - 1), (t + (_NB - 1)) % _NB)
            pltpu.sync_copy(
                out_v, out_hbm.at[pl.ds((wid * _QW + qi) * _CB, _CB)]
            )
            return carry

        lax.fori_loop(0, _QW, chunk, 0)

    return k(g_index, tables_flat)


def kernel(index, offset, tables):
    del offset
    g_index = index + (jnp.arange(_N_TABLES, dtype=jnp.int32) * _VOCAB)[:, None]
    g_index = g_index.reshape(_N_TABLES, _NQ, _GR).transpose(1, 0, 2)
    tables_flat = tables.reshape(_N_TABLES * _VOCAB, _DIM)
    return _sc_embedding_bag(g_index, tables_flat)


# speedup vs baseline: 1.0624x; 1.0007x over previous
import functools

import jax
import jax.numpy as jnp
from jax import lax
from jax.experimental import pallas as pl
from jax.experimental.pallas import tpu as pltpu
from jax.experimental.pallas import tpu_sc as plsc

_N_TABLES = 26
_VOCAB = 100000
_DIM = 64
_BATCH = 4096
_MH = 3

_NC, _NS, _L = 2, 16, 16
_NW = _NC * _NS
_CB = 32
_NQ = _BATCH // _CB
_QW = _NQ // _NW
_GR = _CB * _MH
_NB = 8


def _sc_embedding_bag(g_index, tables_flat):
    mesh = plsc.VectorSubcoreMesh(
        core_axis_name="c", subcore_axis_name="s",
        num_cores=_NC, num_subcores=_NS,
    )

    @functools.partial(
        pl.kernel,
        out_type=jax.ShapeDtypeStruct((_BATCH, _N_TABLES, _DIM), jnp.float32),
        mesh=mesh,
        compiler_params=pltpu.CompilerParams(use_tc_tiling_on_sc=False),
        scratch_types=[
            pltpu.VMEM((_N_TABLES, _GR), jnp.int32),
            pltpu.VMEM((_NB, _GR, _DIM), jnp.float32),
            pltpu.VMEM((_CB, _N_TABLES, _DIM), jnp.float32),
            pltpu.SemaphoreType.DMA((_NB,)),
        ],
    )
    def k(idx_hbm, tbl_hbm, out_hbm, idx_v, rows_v, out_v, sem):
        wid = lax.axis_index("s") * _NC + lax.axis_index("c")

        def gather(t, buf):
            pltpu.async_copy(
                tbl_hbm.at[idx_v.at[t]], rows_v.at[buf], sem.at[buf]
            )

        def drain(buf):
            pltpu.make_async_copy(
                tbl_hbm.at[idx_v.at[0]], rows_v.at[buf], sem.at[buf]
            ).wait()

        def pool(t, buf):
            @plsc.parallel_loop(0, _CB)
            def bag(b):
                r = b * _MH
                for kk in range(_DIM // _L):
                    sl = pl.ds(kk * _L, _L)
                    out_v[b, t, sl] = (
                        rows_v[buf, r, sl]
                        + rows_v[buf, r + 1, sl]
                        + rows_v[buf, r + 2, sl]
                    )

        def chunk(qi, carry):
            pltpu.sync_copy(idx_hbm.at[wid * _QW + qi], idx_v)
            for t in range(_NB - 1):
                gather(t, t)
            for t in range(_N_TABLES):
                buf = t % _NB
                drain(buf)
                pool(t, buf)
                if t + (_NB - 1) < _N_TABLES:
                    gather(t + (_NB - 1), (t + (_NB - 1)) % _NB)
            pltpu.sync_copy(
                out_v, out_hbm.at[pl.ds((wid * _QW + qi) * _CB, _CB)]
            )
            return carry

        lax.fori_loop(0, _QW, chunk, 0)

    return k(g_index, tables_flat)


def kernel(index, offset, tables):
    del offset
    g_index = index + (jnp.arange(_N_TABLES, dtype=jnp.int32) * _VOCAB)[:, None]
    g_index = g_index.reshape(_N_TABLES, _NQ, _GR).transpose(1, 0, 2)
    tables_flat = tables.reshape(_N_TABLES * _VOCAB, _DIM)
    return _sc_embedding_bag(g_index, tables_flat)
